# Initial kernel scaffold; baseline (speedup 1.0000x reference)
import jax
import jax.numpy as jnp
from jax.experimental import pallas as pl


def kernel(x, router_w, router_b, expert_w, expert_b):
    raise NotImplementedError("write your pallas kernel here")


# speedup vs baseline: 2.3697x; 2.3697x over previous
import functools

import jax
import jax.numpy as jnp
from jax.experimental import pallas as pl
from jax.experimental.pallas import tpu as pltpu

NUM_EXPERTS = 16
TOP_K = 2
D_MODEL = 768
T = 2048


def _moe_kernel(x_ref, rw_ref, rb_ref, ew_ref, eb_ref, out_ref, coeff_ref):
    e = pl.program_id(0)

    @pl.when(e == 0)
    def _routing():
        x = x_ref[...]
        logits = jnp.dot(x, rw_ref[...], preferred_element_type=jnp.float32)
        logits = logits + rb_ref[...]
        iota = jax.lax.broadcasted_iota(jnp.int32, logits.shape, 1)
        m1 = jnp.max(logits, axis=1, keepdims=True)
        is1 = logits >= m1
        idx1 = jnp.min(jnp.where(is1, iota, NUM_EXPERTS), axis=1, keepdims=True)
        oh1 = iota == idx1
        masked = jnp.where(oh1, -1e30, logits)
        m2 = jnp.max(masked, axis=1, keepdims=True)
        is2 = masked >= m2
        idx2 = jnp.min(jnp.where(is2, iota, NUM_EXPERTS), axis=1, keepdims=True)
        oh2 = iota == idx2
        g1 = 1.0 / (1.0 + jnp.exp(m2 - m1))
        g2 = 1.0 - g1
        coeff = jnp.where(oh1, g1, 0.0) + jnp.where(oh2, g2, 0.0)
        coeff_ref[...] = coeff
        out_ref[...] = jnp.dot(coeff, eb_ref[...],
                               preferred_element_type=jnp.float32)

    coeff = coeff_ref[...]
    lane = jax.lax.broadcasted_iota(jnp.int32, coeff.shape, 1)
    ce = jnp.sum(jnp.where(lane == e, coeff, 0.0), axis=1, keepdims=True)
    xs = (ce * x_ref[...]).astype(jnp.bfloat16)
    w = ew_ref[0].astype(jnp.bfloat16)
    out_ref[...] += jnp.dot(xs, w, preferred_element_type=jnp.float32)


@jax.jit
def kernel(x, router_w, router_b, expert_w, expert_b):
    rb2 = router_b.reshape(1, NUM_EXPERTS)
    grid = (NUM_EXPERTS,)
    return pl.pallas_call(
        _moe_kernel,
        grid=grid,
        in_specs=[
            pl.BlockSpec((T, D_MODEL), lambda e: (0, 0)),
            pl.BlockSpec((D_MODEL, NUM_EXPERTS), lambda e: (0, 0)),
            pl.BlockSpec((1, NUM_EXPERTS), lambda e: (0, 0)),
            pl.BlockSpec((1, D_MODEL, D_MODEL), lambda e: (e, 0, 0)),
            pl.BlockSpec((NUM_EXPERTS, D_MODEL), lambda e: (0, 0)),
        ],
        out_specs=pl.BlockSpec((T, D_MODEL), lambda e: (0, 0)),
        out_shape=jax.ShapeDtypeStruct((T, D_MODEL), jnp.float32),
        scratch_shapes=[pltpu.VMEM((T, NUM_EXPERTS), jnp.float32)],
        compiler_params=pltpu.CompilerParams(
            dimension_semantics=("arbitrary",)),
    )(x, router_w, rb2, expert_w, expert_b)


# speedup vs baseline: 2.3862x; 1.0069x over previous
import functools

import jax
import jax.numpy as jnp
from jax.experimental import pallas as pl
from jax.experimental.pallas import tpu as pltpu

NUM_EXPERTS = 16
TOP_K = 2
D_MODEL = 768
T = 2048


def _moe_kernel(x_ref, rw_ref, rb_ref, ew_ref, eb_ref, out_ref, coeff_ref,
                xbf_ref):
    e = pl.program_id(0)

    @pl.when(e == 0)
    def _routing():
        x = x_ref[...]
        xbf_ref[...] = x.astype(jnp.bfloat16)
        logits = jnp.dot(x, rw_ref[...], preferred_element_type=jnp.float32)
        logits = logits + rb_ref[...]
        iota = jax.lax.broadcasted_iota(jnp.int32, logits.shape, 1)
        m1 = jnp.max(logits, axis=1, keepdims=True)
        is1 = logits >= m1
        idx1 = jnp.min(jnp.where(is1, iota, NUM_EXPERTS), axis=1, keepdims=True)
        oh1 = iota == idx1
        masked = jnp.where(oh1, -1e30, logits)
        m2 = jnp.max(masked, axis=1, keepdims=True)
        is2 = masked >= m2
        idx2 = jnp.min(jnp.where(is2, iota, NUM_EXPERTS), axis=1, keepdims=True)
        oh2 = iota == idx2
        g1 = 1.0 / (1.0 + jnp.exp(m2 - m1))
        g2 = 1.0 - g1
        coeff = jnp.where(oh1, g1, 0.0) + jnp.where(oh2, g2, 0.0)
        coeff_ref[...] = coeff
        out_ref[...] = jnp.dot(coeff, eb_ref[...],
                               preferred_element_type=jnp.float32)

    coeff = coeff_ref[...]
    lane = jax.lax.broadcasted_iota(jnp.int32, coeff.shape, 1)
    ce = jnp.sum(jnp.where(lane == e, coeff, 0.0), axis=1, keepdims=True)
    w = ew_ref[0].astype(jnp.bfloat16)
    y = jnp.dot(xbf_ref[...], w, preferred_element_type=jnp.float32)
    out_ref[...] += ce * y


@jax.jit
def kernel(x, router_w, router_b, expert_w, expert_b):
    rb2 = router_b.reshape(1, NUM_EXPERTS)
    grid = (NUM_EXPERTS,)
    return pl.pallas_call(
        _moe_kernel,
        grid=grid,
        in_specs=[
            pl.BlockSpec((T, D_MODEL), lambda e: (0, 0)),
            pl.BlockSpec((D_MODEL, NUM_EXPERTS), lambda e: (0, 0)),
            pl.BlockSpec((1, NUM_EXPERTS), lambda e: (0, 0)),
            pl.BlockSpec((1, D_MODEL, D_MODEL), lambda e: (e, 0, 0)),
            pl.BlockSpec((NUM_EXPERTS, D_MODEL), lambda e: (0, 0)),
        ],
        out_specs=pl.BlockSpec((T, D_MODEL), lambda e: (0, 0)),
        out_shape=jax.ShapeDtypeStruct((T, D_MODEL), jnp.float32),
        scratch_shapes=[pltpu.VMEM((T, NUM_EXPERTS), jnp.float32),
                        pltpu.VMEM((T, D_MODEL), jnp.bfloat16)],
        compiler_params=pltpu.CompilerParams(
            dimension_semantics=("arbitrary",)),
    )(x, router_w, rb2, expert_w, expert_b)
